# Initial kernel scaffold; baseline (speedup 1.0000x reference)
import jax
import jax.numpy as jnp
from jax.experimental import pallas as pl


def kernel(x, edge_index, edge_attr, batch, W1, root1, b1, W2, root2, b2, fc1_w, fc1_b, fc2_w, fc2_b):
    raise NotImplementedError("write your pallas kernel here")


# speedup vs baseline: 7.2259x; 7.2259x over previous
import functools

import jax
import jax.numpy as jnp
from jax import lax
from jax.experimental import pallas as pl
from jax.experimental.pallas import tpu as pltpu
from jax.experimental.pallas import tpu_sc as plsc

KS = 5
K = KS * KS
N_NODES = 100000
N_EDGES = 1600000
N_GRAPHS = 64

NW = 32
EW = N_EDGES // NW
SH = 7
BN = 1 << SH
BK = 800
SLOTS = BK // NW
NP = BK * BN
CH = 2000
NWIN = EW // CH
CH3 = 256
EP = N_EDGES + CH3

NODE_BLK = 2048
N_BLKS = NP // NODE_BLK

_SC_PARAMS = pltpu.CompilerParams(needs_layout_passes=False,
                                  use_tc_tiling_on_sc=False)


@functools.cache
def _mesh():
    return plsc.VectorSubcoreMesh(core_axis_name="c", subcore_axis_name="s")


def _wid():
    return lax.axis_index("s") * 2 + lax.axis_index("c")


@functools.cache
def _build_hist():
    return functools.partial(
        pl.kernel, mesh=_mesh(), compiler_params=_SC_PARAMS,
        out_type=jax.ShapeDtypeStruct((NW * BK,), jnp.float32),
        scratch_types=[
            pltpu.VMEM((CH,), jnp.int32),
            pltpu.VMEM((BK,), jnp.float32),
        ],
    )(_hist_body)


def _hist_body(dst_hbm, counts_hbm, dbuf, cnt):
    w = _wid()
    zf = jnp.zeros((16,), jnp.float32)
    ones = jnp.ones((16,), jnp.float32)

    def zbody(i, _):
        cnt[pl.ds(i * 16, 16)] = zf
        return 0

    lax.fori_loop(0, BK // 16, zbody, 0)

    def win(j, _):
        pltpu.sync_copy(dst_hbm.at[pl.ds(w * EW + j * CH, CH)], dbuf)

        def vb(v, _):
            d = dbuf[pl.ds(v * 16, 16)]
            b = lax.shift_right_logical(d, SH)
            plsc.addupdate_scatter(cnt, [b], ones)
            return 0

        lax.fori_loop(0, CH // 16, vb, 0)
        return 0

    lax.fori_loop(0, NWIN, win, 0)
    pltpu.sync_copy(cnt, counts_hbm.at[pl.ds(w * BK, BK)])


@functools.cache
def _build_place():
    return functools.partial(
        pl.kernel, mesh=_mesh(), compiler_params=_SC_PARAMS,
        out_type=[
            jax.ShapeDtypeStruct((EP, 8), jnp.int32),
            jax.ShapeDtypeStruct((BK + 16,), jnp.int32),
        ],
        scratch_types=[
            pltpu.VMEM((NW * BK,), jnp.float32),
            pltpu.VMEM((BK,), jnp.float32),
            pltpu.VMEM((BK + 16,), jnp.int32),
            pltpu.VMEM((CH,), jnp.int32),
            pltpu.VMEM((CH,), jnp.int32),
            pltpu.VMEM((2 * CH,), jnp.float32),
            pltpu.VMEM((CH, 8), jnp.int32),
            pltpu.VMEM((CH + 16,), jnp.int32),
            pltpu.VMEM((100,), jnp.int32),
            pltpu.VMEM((48,), jnp.int32),
            pltpu.VMEM((16,), jnp.int32),
            pltpu.SemaphoreType.DMA,
        ],
    )(_place_body)


def _place_body(src_hbm, dst_hbm, attr_hbm, counts_hbm, rec_hbm, start_hbm,
                cbuf, cntv, startv, sbuf, dbuf, abuf, recstage, posbuf,
                posfull, tmp, pos16, sem):
    w = _wid()
    iot = lax.iota(jnp.int32, 16)
    pltpu.sync_copy(counts_hbm, cbuf)

    def pbody(i, carry):
        t = jnp.zeros((16,), jnp.float32)
        p = jnp.zeros((16,), jnp.float32)
        for ti in range(NW):
            row = cbuf[pl.ds(ti * BK + i * 16, 16)]
            t = t + row
            m = jnp.where(ti < w, 1.0, 0.0)
            p = p + row * m
        c = plsc.cumsum(t)
        excl = c - t + carry
        cntv[pl.ds(i * 16, 16)] = excl + p
        startv[pl.ds(i * 16, 16)] = jnp.astype(excl, jnp.int32)
        return carry + c[15]

    lax.fori_loop(0, BK // 16, pbody, 0.0)
    startv[pl.ds(BK, 16)] = jnp.zeros((16,), jnp.int32) + N_EDGES

    @pl.when(w == 0)
    def _():
        pltpu.sync_copy(startv, start_hbm)

    patt_a = (lax.shift_right_logical(iot, 1) * 8 + 2 + (iot & 1))
    ones = jnp.ones((16,), jnp.float32)

    def win(j, _):
        base = w * EW + j * CH
        pltpu.sync_copy(src_hbm.at[pl.ds(base, CH)], sbuf)
        pltpu.sync_copy(dst_hbm.at[pl.ds(base, CH)], dbuf)
        pltpu.sync_copy(attr_hbm.at[pl.ds(base * 2, CH * 2)], abuf)

        def ab(v, _):
            fv = abuf[pl.ds(v * 16, 16)]
            a = patt_a + 64 * v
            plsc.store_scatter(recstage, [lax.shift_right_logical(a, 3), a & 7],
                               plsc.bitcast(fv, jnp.int32))
            return 0

        lax.fori_loop(0, 2 * CH // 16, ab, 0)

        def sb(v, _):
            sv = sbuf[pl.ds(v * 16, 16)]
            plsc.store_scatter(recstage, [v * 16 + iot,
                                          jnp.zeros((16,), jnp.int32)], sv)
            return 0

        lax.fori_loop(0, CH // 16, sb, 0)

        def db(v, _):
            d = dbuf[pl.ds(v * 16, 16)]
            plsc.store_scatter(recstage, [v * 16 + iot,
                                          jnp.zeros((16,), jnp.int32) + 1], d)
            b = lax.shift_right_logical(d, SH)
            ks, lane = plsc.sort_key_val(b, iot)
            tmp[pl.ds(0, 16)] = jnp.zeros((16,), jnp.int32) - 1
            tmp[pl.ds(1, 16)] = ks
            prv = tmp[pl.ds(0, 16)]
            is_first = ks != prv
            rank = iot - plsc.cummax(jnp.where(is_first, iot, 0))
            cur = plsc.load_gather(cntv, [ks])
            pos_s = jnp.astype(cur, jnp.int32) + rank
            plsc.store_scatter(pos16, [lane], pos_s)
            plsc.addupdate_scatter(cntv, [b], ones)
            posbuf[pl.ds(v * 16, 16)] = pos16[...]
            return 0

        lax.fori_loop(0, CH // 16, db, 0)

        def dma(r, _):
            for k in range(6):
                posfull[pl.ds(k * 16, 16)] = posbuf[pl.ds(r * 100 + k * 16, 16)]
            posfull[pl.ds(84, 16)] = posbuf[pl.ds(r * 100 + 84, 16)]
            cp = pltpu.async_copy(recstage.at[pl.ds(r * 100, 100)],
                                  rec_hbm.at[posfull], sem)
            cp.wait()
            return 0

        lax.fori_loop(0, 20, dma, 0)
        return 0

    lax.fori_loop(0, NWIN, win, 0)


def _make_accum(cin, with_deg):
    kc = K * cin
    out_types = [jax.ShapeDtypeStruct((NP * kc,), jnp.float32)]
    if with_deg:
        out_types.append(jax.ShapeDtypeStruct((NP,), jnp.float32))
    scratch = [
        pltpu.VMEM((BN * kc,), jnp.float32),
        pltpu.VMEM((BK + 16,), jnp.int32),
        pltpu.VMEM((CH3 * 8,), jnp.int32),
        pltpu.VMEM((CH3,), jnp.int32),
        pltpu.VMEM((CH3, cin), jnp.float32),
        pltpu.VMEM((4 * CH3,), jnp.float32),
        pltpu.VMEM((4 * CH3,), jnp.int32),
        pltpu.SemaphoreType.DMA,
    ]
    if with_deg:
        scratch.insert(1, pltpu.VMEM((BN,), jnp.float32))

    def body(rec_hbm, start_hbm, x_hbm, *refs):
        if with_deg:
            (acc_hbm, deg_hbm, acc, degv, startv, recbuf, srcv, xsb, wv, rv,
             sem) = refs
        else:
            (acc_hbm, acc, startv, recbuf, srcv, xsb, wv, rv, sem) = refs
            degv = None
        w = _wid()
        iot = lax.iota(jnp.int32, 16)
        zf = jnp.zeros((16,), jnp.float32)
        pltpu.sync_copy(start_hbm, startv)

        def extract(bb):
            off = bb & ~15
            vv = startv[pl.ds(off, 16)]
            return jnp.sum(jnp.where(iot == (bb - off), vv, 0))

        def slot_body(slot, _carry):
            b = slot * NW + w
            e0 = extract(b)
            e1 = extract(b + 1)

            def zb(i, _):
                for u in range(8):
                    acc[pl.ds(i * 128 + u * 16, 16)] = zf
                return 0

            lax.fori_loop(0, BN * kc // 128, zb, 0)
            if with_deg:
                def zd(i, _):
                    degv[pl.ds(i * 16, 16)] = zf
                    return 0

                lax.fori_loop(0, BN // 16, zd, 0)

            ba8 = lax.shift_right_logical(e0, 3)
            basea = ba8 * 8
            nwin = lax.shift_right_logical(e1 - basea + CH3 - 1, 8)

            def win(j, _):
                wb = basea + j * CH3
                pltpu.sync_copy(
                    rec_hbm.at[pl.ds(ba8 * 64 + j * (CH3 * 8), CH3 * 8)],
                    recbuf)

                def vb(v, _):
                    fidx = 128 * v + 8 * iot
                    s = plsc.load_gather(recbuf, [fidx])
                    d = plsc.load_gather(recbuf, [fidx + 1])
                    a0 = plsc.bitcast(plsc.load_gather(recbuf, [fidx + 2]),
                                      jnp.float32)
                    a1 = plsc.bitcast(plsc.load_gather(recbuf, [fidx + 3]),
                                      jnp.float32)
                    ge = wb + v * 16 + iot
                    valid = (ge >= e0) & (ge < e1)
                    wm = jnp.where(valid, 1.0, 0.0)
                    s = jnp.clip(s, 0, N_NODES - 1)
                    srcv[pl.ds(v * 16, 16)] = s
                    dr = jnp.clip(d - b * BN, 0, BN - 1)
                    v0 = a0 * (KS - 1.0)
                    v1 = a1 * (KS - 1.0)
                    i0 = jnp.clip(jnp.astype(v0, jnp.int32), 0, KS - 2)
                    i1 = jnp.clip(jnp.astype(v1, jnp.int32), 0, KS - 2)
                    f0 = v0 - jnp.astype(i0, jnp.float32)
                    f1 = v1 - jnp.astype(i1, jnp.float32)
                    g0 = 1.0 - f0
                    g1 = 1.0 - f1
                    zf16 = jnp.zeros((16,), jnp.float32)
                    rbase = (dr * K + i0 + KS * i1) * cin
                    wv[pl.ds(v * 16, 16)] = jnp.where(valid, g0 * g1, zf16)
                    rv[pl.ds(v * 16, 16)] = rbase
                    wv[pl.ds(CH3 + v * 16, 16)] = jnp.where(valid, f0 * g1, zf16)
                    rv[pl.ds(CH3 + v * 16, 16)] = rbase + cin
                    wv[pl.ds(2 * CH3 + v * 16, 16)] = jnp.where(valid, g0 * f1,
                                                                zf16)
                    rv[pl.ds(2 * CH3 + v * 16, 16)] = rbase + KS * cin
                    wv[pl.ds(3 * CH3 + v * 16, 16)] = jnp.where(valid, f0 * f1,
                                                                zf16)
                    rv[pl.ds(3 * CH3 + v * 16, 16)] = rbase + (KS + 1) * cin
                    if with_deg:
                        plsc.addupdate_scatter(degv, [dr], wm)
                    return 0

                lax.fori_loop(0, CH3 // 16, vb, 0)

                cp0 = pltpu.async_copy(x_hbm.at[srcv.at[pl.ds(0, 128)]],
                                       xsb.at[pl.ds(0, 128)], sem)
                cp1 = pltpu.async_copy(x_hbm.at[srcv.at[pl.ds(128, 128)]],
                                       xsb.at[pl.ds(128, 128)], sem)
                cp0.wait()
                cp1.wait()

                def gb(g, _):
                    w0 = wv[pl.ds(g * 16, 16)]
                    w1 = wv[pl.ds(CH3 + g * 16, 16)]
                    w2 = wv[pl.ds(2 * CH3 + g * 16, 16)]
                    w3 = wv[pl.ds(3 * CH3 + g * 16, 16)]
                    r0 = rv[pl.ds(g * 16, 16)]
                    r1 = rv[pl.ds(CH3 + g * 16, 16)]
                    r2 = rv[pl.ds(2 * CH3 + g * 16, 16)]
                    r3 = rv[pl.ds(3 * CH3 + g * 16, 16)]
                    for l in range(16):
                        e = g * 16 + l
                        for half in range(cin // 16):
                            ho = half * 16
                            xs = xsb[e, pl.ds(ho, 16)]
                            plsc.addupdate(acc.at[pl.ds(r0[l] + ho, 16)],
                                           xs * w0[l])
                            plsc.addupdate(acc.at[pl.ds(r1[l] + ho, 16)],
                                           xs * w1[l])
                            plsc.addupdate(acc.at[pl.ds(r2[l] + ho, 16)],
                                           xs * w2[l])
                            plsc.addupdate(acc.at[pl.ds(r3[l] + ho, 16)],
                                           xs * w3[l])
                    return 0

                lax.fori_loop(0, CH3 // 16, gb, 0)
                return 0

            lax.fori_loop(0, nwin, win, 0)
            pltpu.sync_copy(acc, acc_hbm.at[pl.ds(b * (BN * kc), BN * kc)])
            if with_deg:
                pltpu.sync_copy(degv, deg_hbm.at[pl.ds(b * BN, BN)])
            return 0

        lax.fori_loop(0, SLOTS, slot_body, 0)

    return functools.partial(
        pl.kernel, mesh=_mesh(), compiler_params=_SC_PARAMS,
        out_type=out_types, scratch_types=scratch)(body)


_accum = functools.cache(_make_accum)


def _dense_body(acc_ref, deg_ref, x_ref, wf_ref, root_ref, b_ref, o_ref):
    acc = acc_ref[...]
    deg = jnp.maximum(deg_ref[...], 1.0)
    o = jnp.dot(acc, wf_ref[...], preferred_element_type=jnp.float32)
    o = o / deg
    o = o + jnp.dot(x_ref[...], root_ref[...], preferred_element_type=jnp.float32)
    o = o + b_ref[...]
    o_ref[...] = jnp.maximum(o, 0.0)


def _dense_layer(acc, deg, x, Wf, root, b):
    n, kcin = acc.shape
    cin = x.shape[1]
    cout = Wf.shape[1]
    return pl.pallas_call(
        _dense_body,
        grid=(N_BLKS,),
        in_specs=[
            pl.BlockSpec((NODE_BLK, kcin), lambda i: (i, 0)),
            pl.BlockSpec((NODE_BLK, 1), lambda i: (i, 0)),
            pl.BlockSpec((NODE_BLK, cin), lambda i: (i, 0)),
            pl.BlockSpec((kcin, cout), lambda i: (0, 0)),
            pl.BlockSpec((cin, cout), lambda i: (0, 0)),
            pl.BlockSpec((1, cout), lambda i: (0, 0)),
        ],
        out_specs=pl.BlockSpec((NODE_BLK, cout), lambda i: (i, 0)),
        out_shape=jax.ShapeDtypeStruct((n, cout), jnp.float32),
    )(acc, deg.reshape(n, 1), x, Wf, root, b.reshape(1, cout))


def _pool_mlp_body(h_ref, batch_ref, fc1w_ref, fc1b_ref, fc2w_ref, fc2b_ref,
                   o_ref, gsum, gcnt):
    i = pl.program_id(0)

    @pl.when(i == 0)
    def _init():
        gsum[...] = jnp.zeros_like(gsum)
        gcnt[...] = jnp.zeros_like(gcnt)

    bvec = batch_ref[...][:, 0]
    gids = jax.lax.broadcasted_iota(jnp.int32, (N_GRAPHS, NODE_BLK), 0)
    mask = (bvec[None, :] == gids).astype(jnp.float32)
    gsum[...] += jnp.dot(mask, h_ref[...], preferred_element_type=jnp.float32)
    gcnt[...] += jnp.sum(mask, axis=1, keepdims=True)

    @pl.when(i == N_BLKS - 1)
    def _fin():
        g = gsum[...] / jnp.maximum(gcnt[...], 1.0)
        z = jnp.dot(g, fc1w_ref[...], preferred_element_type=jnp.float32)
        z = jnp.maximum(z + fc1b_ref[...], 0.0)
        z = jnp.dot(z, fc2w_ref[...], preferred_element_type=jnp.float32)
        z = z + fc2b_ref[...]
        m = jnp.max(z, axis=1, keepdims=True)
        e = z - m
        lse = jnp.log(jnp.sum(jnp.exp(e), axis=1, keepdims=True))
        o_ref[...] = e - lse


def _pool_mlp(h, batch, fc1_w, fc1_b, fc2_w, fc2_b):
    hdim = h.shape[1]
    out = fc2_w.shape[1]
    return pl.pallas_call(
        _pool_mlp_body,
        grid=(N_BLKS,),
        in_specs=[
            pl.BlockSpec((NODE_BLK, hdim), lambda i: (i, 0)),
            pl.BlockSpec((NODE_BLK, 1), lambda i: (i, 0)),
            pl.BlockSpec((hdim, 128), lambda i: (0, 0)),
            pl.BlockSpec((1, 128), lambda i: (0, 0)),
            pl.BlockSpec((128, out), lambda i: (0, 0)),
            pl.BlockSpec((1, out), lambda i: (0, 0)),
        ],
        out_specs=pl.BlockSpec((N_GRAPHS, out), lambda i: (0, 0)),
        out_shape=jax.ShapeDtypeStruct((N_GRAPHS, out), jnp.float32),
        scratch_shapes=[
            pltpu.VMEM((N_GRAPHS, hdim), jnp.float32),
            pltpu.VMEM((N_GRAPHS, 1), jnp.float32),
        ],
    )(h, batch.reshape(-1, 1), fc1_w, fc1_b.reshape(1, 128), fc2_w,
      fc2_b.reshape(1, out))


def kernel(x, edge_index, edge_attr, batch, W1, root1, b1, W2, root2, b2,
           fc1_w, fc1_b, fc2_w, fc2_b):
    src = edge_index[0]
    dst = edge_index[1]
    attr_flat = edge_attr.reshape(2 * N_EDGES)
    x_pad = jnp.pad(x, ((0, NP - N_NODES), (0, 0)))
    batch_pad = jnp.pad(batch, (0, NP - N_NODES), constant_values=127)

    counts = _build_hist()(dst)
    rec, start = _build_place()(src, dst, attr_flat, counts)
    rec_flat = rec.reshape(EP * 8)

    acc1f, deg = _accum(16, True)(rec_flat, start, x_pad)
    h1 = _dense_layer(acc1f.reshape(NP, K * 16), deg, x_pad,
                      W1.reshape(K * 16, 32), root1, b1)

    (acc2f,) = _accum(32, False)(rec_flat, start, h1)
    h2 = _dense_layer(acc2f.reshape(NP, K * 32), deg, h1,
                      W2.reshape(K * 32, 64), root2, b2)

    return _pool_mlp(h2, batch_pad, fc1_w, fc1_b, fc2_w, fc2_b)
